# Initial kernel scaffold; baseline (speedup 1.0000x reference)
import jax
import jax.numpy as jnp
from jax.experimental import pallas as pl


def kernel(mg_emb, var_indices, patch_idx, W, b):
    raise NotImplementedError("write your pallas kernel here")


# speedup vs baseline: 2.2867x; 2.2867x over previous
import functools

import jax
import jax.numpy as jnp
from jax import lax
from jax.experimental import pallas as pl
from jax.experimental.pallas import tpu as pltpu
from jax.experimental.pallas import tpu_sc as plsc

_CHUNK = 128
_NBUF = 8


@functools.lru_cache(maxsize=None)
def _make_gather(n_table_rows: int, n_rows: int, row_width: int):
    info = plsc.get_sparse_core_info()
    nc, ns = info.num_cores, info.num_subcores
    nw = nc * ns
    assert n_rows % (nw * _CHUNK) == 0
    rows_per_w = n_rows // nw
    nch = rows_per_w // _CHUNK
    ngroups = nch // _NBUF
    assert ngroups * _NBUF == nch
    mesh = plsc.VectorSubcoreMesh(core_axis_name="c", subcore_axis_name="s")

    @functools.partial(
        pl.kernel,
        mesh=mesh,
        out_type=jax.ShapeDtypeStruct((n_rows, row_width), jnp.float32),
        scratch_types=[
            pltpu.VMEM((nch, _CHUNK), jnp.int32),
            pltpu.VMEM((_NBUF, _CHUNK, row_width), jnp.float32),
            pltpu.SemaphoreType.DMA,
            pltpu.SemaphoreType.DMA,
        ],
        compiler_params=pltpu.CompilerParams(use_tc_tiling_on_sc=False),
    )
    def gather(table_hbm, idx_hbm, out_hbm, idx_v, rows_v, gsem, wsem):
        wid = lax.axis_index("s") * nc + lax.axis_index("c")
        pltpu.sync_copy(idx_hbm.at[pl.ds(wid * nch, nch)], idx_v)
        base = wid * rows_per_w

        def group(g, _):
            j0 = g * _NBUF
            for bb in range(_NBUF):
                pltpu.async_copy(table_hbm.at[idx_v.at[j0 + bb]], rows_v.at[bb], gsem)
            for bb in range(_NBUF):
                pltpu.make_async_copy(table_hbm.at[idx_v.at[j0 + bb]], rows_v.at[bb], gsem).wait()
            for bb in range(_NBUF):
                pltpu.async_copy(
                    rows_v.at[bb],
                    out_hbm.at[pl.ds(base + (j0 + bb) * _CHUNK, _CHUNK)],
                    wsem,
                )
            for bb in range(_NBUF):
                pltpu.make_async_copy(
                    rows_v.at[bb],
                    out_hbm.at[pl.ds(base + (j0 + bb) * _CHUNK, _CHUNK)],
                    wsem,
                ).wait()
            return 0

        lax.fori_loop(0, ngroups, group, 0)

    return gather


def _proj_body(x_ref, w_ref, b_ref, o_ref):
    o_ref[...] = (
        jnp.dot(x_ref[...], w_ref[...], preferred_element_type=jnp.float32)
        + b_ref[...]
    )


@functools.lru_cache(maxsize=None)
def _make_proj(n_rows: int, c: int, d: int, blk: int = 8192):
    assert n_rows % blk == 0
    return pl.pallas_call(
        _proj_body,
        grid=(n_rows // blk,),
        in_specs=[
            pl.BlockSpec((blk, c), lambda i: (i, 0)),
            pl.BlockSpec((c, d), lambda i: (0, 0)),
            pl.BlockSpec((1, d), lambda i: (0, 0)),
        ],
        out_specs=pl.BlockSpec((blk, d), lambda i: (i, 0)),
        out_shape=jax.ShapeDtypeStruct((n_rows, d), jnp.float32),
    )


def kernel(mg_emb, var_indices, patch_idx, W, b):
    n_var, n_nodes, C = mg_emb.shape
    B, V = var_indices.shape
    P = patch_idx.shape[-1]
    D = W.shape[-1]
    n_rows = B * V * P

    table = mg_emb.reshape(n_var * n_nodes, C)
    idx = (
        var_indices.astype(jnp.int32)[:, :, None] * n_nodes
        + patch_idx.astype(jnp.int32)[:, None, :]
    ).reshape(n_rows // _CHUNK, _CHUNK)

    gathered = _make_gather(n_var * n_nodes, n_rows, C)(table, idx)
    out = _make_proj(n_rows, C, D)(gathered, W, b.reshape(1, D))
    return out.reshape(B, V, 1, P, D)


# speedup vs baseline: 3.8988x; 1.7050x over previous
import functools

import jax
import jax.numpy as jnp
from jax import lax
from jax.experimental import pallas as pl
from jax.experimental.pallas import tpu as pltpu
from jax.experimental.pallas import tpu_sc as plsc

_CHUNK = 128
_NBUF = 8


@functools.lru_cache(maxsize=None)
def _make_gather(n_table_rows: int, n_rows: int, row_width: int):
    info = plsc.get_sparse_core_info()
    nc, ns = info.num_cores, info.num_subcores
    nw = nc * ns
    assert n_rows % (nw * _CHUNK) == 0
    rows_per_w = n_rows // nw
    nch = rows_per_w // _CHUNK
    ngroups = nch // _NBUF
    assert ngroups * _NBUF == nch
    pack = 128 // row_width
    band = rows_per_w // pack
    chunks_per_u = band // _CHUNK
    mesh = plsc.VectorSubcoreMesh(core_axis_name="c", subcore_axis_name="s")

    @functools.partial(
        pl.kernel,
        mesh=mesh,
        out_type=jax.ShapeDtypeStruct((n_rows // pack, 128), jnp.float32),
        scratch_types=[
            pltpu.VMEM((nch, _CHUNK), jnp.int32),
            pltpu.VMEM((_NBUF, _CHUNK, row_width), jnp.float32),
            pltpu.SemaphoreType.DMA,
            pltpu.SemaphoreType.DMA,
        ],
        compiler_params=pltpu.CompilerParams(use_tc_tiling_on_sc=False),
    )
    def gather(table_hbm, idx_hbm, out_hbm, idx_v, rows_v, gsem, wsem):
        wid = lax.axis_index("s") * nc + lax.axis_index("c")
        pltpu.sync_copy(idx_hbm.at[pl.ds(wid * nch, nch)], idx_v)
        base = wid * band

        def group(g, _):
            j0 = g * _NBUF
            for bb in range(_NBUF):
                pltpu.async_copy(table_hbm.at[idx_v.at[j0 + bb]], rows_v.at[bb], gsem)
            for bb in range(_NBUF):
                pltpu.make_async_copy(table_hbm.at[idx_v.at[j0 + bb]], rows_v.at[bb], gsem).wait()
            for bb in range(_NBUF):
                c = j0 + bb
                dst = out_hbm.at[
                    pl.ds(base + (c % chunks_per_u) * _CHUNK, _CHUNK),
                    pl.ds((c // chunks_per_u) * row_width, row_width),
                ]
                pltpu.async_copy(rows_v.at[bb], dst, wsem)
            for bb in range(_NBUF):
                c = j0 + bb
                dst = out_hbm.at[
                    pl.ds(base + (c % chunks_per_u) * _CHUNK, _CHUNK),
                    pl.ds((c // chunks_per_u) * row_width, row_width),
                ]
                pltpu.make_async_copy(rows_v.at[bb], dst, wsem).wait()
            return 0

        lax.fori_loop(0, ngroups, group, 0)

    return gather


def _proj_body(x_ref, wt_ref, b_ref, o_ref):
    pack = x_ref.shape[1] // wt_ref.shape[1]
    c = wt_ref.shape[1]
    for u in range(pack):
        xu = x_ref[:, u * c:(u + 1) * c]
        ot = jax.lax.dot_general(
            wt_ref[...], xu,
            dimension_numbers=(((1,), (1,)), ((), ())),
            preferred_element_type=jnp.float32,
        )
        o_ref[0, u] = ot + b_ref[...]


@functools.lru_cache(maxsize=None)
def _make_proj(nw: int, band: int, c: int, d: int):
    pack = 128 // c
    return pl.pallas_call(
        _proj_body,
        grid=(nw,),
        in_specs=[
            pl.BlockSpec((band, 128), lambda i: (i, 0)),
            pl.BlockSpec((d, c), lambda i: (0, 0)),
            pl.BlockSpec((d, 1), lambda i: (0, 0)),
        ],
        out_specs=pl.BlockSpec((1, pack, d, band), lambda i: (i, 0, 0, 0)),
        out_shape=jax.ShapeDtypeStruct((nw, pack, d, band), jnp.float32),
    )


def kernel(mg_emb, var_indices, patch_idx, W, b):
    n_var, n_nodes, C = mg_emb.shape
    B, V = var_indices.shape
    P = patch_idx.shape[-1]
    D = W.shape[-1]
    n_rows = B * V * P

    table = mg_emb.reshape(n_var * n_nodes, C)
    idx = (
        var_indices.astype(jnp.int32)[:, :, None] * n_nodes
        + patch_idx.astype(jnp.int32)[:, None, :]
    ).reshape(n_rows // _CHUNK, _CHUNK)

    packed = _make_gather(n_var * n_nodes, n_rows, C)(table, idx)
    nw = 32
    band = n_rows // nw // (128 // C)
    ot = _make_proj(nw, band, C, D)(packed, W.T, b.reshape(D, 1))
    out = ot.reshape(B, V, 1, D, P)
    return jnp.swapaxes(out, -1, -2)


# speedup vs baseline: 3.9122x; 1.0034x over previous
import functools

import jax
import jax.numpy as jnp
from jax import lax
from jax.experimental import pallas as pl
from jax.experimental.pallas import tpu as pltpu
from jax.experimental.pallas import tpu_sc as plsc

_CHUNK = 128
_NBUF = 8


@functools.lru_cache(maxsize=None)
def _make_gather(n_table_rows: int, n_rows: int, row_width: int):
    info = plsc.get_sparse_core_info()
    nc, ns = info.num_cores, info.num_subcores
    nw = nc * ns
    assert n_rows % (nw * _CHUNK) == 0
    rows_per_w = n_rows // nw
    nch = rows_per_w // _CHUNK
    ngroups = nch // _NBUF
    assert ngroups * _NBUF == nch
    pack = 128 // row_width
    band = rows_per_w // pack
    chunks_per_u = band // _CHUNK
    mesh = plsc.VectorSubcoreMesh(core_axis_name="c", subcore_axis_name="s")

    @functools.partial(
        pl.kernel,
        mesh=mesh,
        out_type=jax.ShapeDtypeStruct((n_rows // pack, 128), jnp.float32),
        scratch_types=[
            pltpu.VMEM((nch, _CHUNK), jnp.int32),
            pltpu.VMEM((_NBUF, _CHUNK, row_width), jnp.float32),
            pltpu.SemaphoreType.DMA,
            pltpu.SemaphoreType.DMA,
        ],
        compiler_params=pltpu.CompilerParams(use_tc_tiling_on_sc=False),
    )
    def gather(table_hbm, idx_hbm, out_hbm, idx_v, rows_v, gsem, wsem):
        wid = lax.axis_index("s") * nc + lax.axis_index("c")
        pltpu.sync_copy(idx_hbm.at[pl.ds(wid * nch, nch)], idx_v)
        base = wid * band

        def group(g, _):
            j0 = g * _NBUF
            for bb in range(_NBUF):
                pltpu.async_copy(table_hbm.at[idx_v.at[j0 + bb]], rows_v.at[bb], gsem)
            for bb in range(_NBUF):
                pltpu.make_async_copy(table_hbm.at[idx_v.at[j0 + bb]], rows_v.at[bb], gsem).wait()
            for bb in range(_NBUF):
                c = j0 + bb
                dst = out_hbm.at[
                    pl.ds(base + (c % chunks_per_u) * _CHUNK, _CHUNK),
                    pl.ds((c // chunks_per_u) * row_width, row_width),
                ]
                pltpu.async_copy(rows_v.at[bb], dst, wsem)
            for bb in range(_NBUF):
                c = j0 + bb
                dst = out_hbm.at[
                    pl.ds(base + (c % chunks_per_u) * _CHUNK, _CHUNK),
                    pl.ds((c // chunks_per_u) * row_width, row_width),
                ]
                pltpu.make_async_copy(rows_v.at[bb], dst, wsem).wait()
            return 0

        lax.fori_loop(0, ngroups, group, 0)

    return gather


def _proj_body(x_ref, wt_ref, b_ref, o_ref):
    pack = x_ref.shape[1] // wt_ref.shape[1]
    c = wt_ref.shape[1]
    for u in range(pack):
        xu = x_ref[:, u * c:(u + 1) * c]
        ot = jax.lax.dot_general(
            wt_ref[...], xu,
            dimension_numbers=(((1,), (1,)), ((), ())),
            preferred_element_type=jnp.float32,
        )
        o_ref[0, u] = ot + b_ref[...]


@functools.lru_cache(maxsize=None)
def _make_proj(nw: int, band: int, c: int, d: int):
    pack = 128 // c
    return pl.pallas_call(
        _proj_body,
        grid=(nw,),
        in_specs=[
            pl.BlockSpec((band, 128), lambda i: (i, 0)),
            pl.BlockSpec((d, c), lambda i: (0, 0)),
            pl.BlockSpec((d, 1), lambda i: (0, 0)),
        ],
        out_specs=pl.BlockSpec((1, pack, d, band), lambda i: (i, 0, 0, 0)),
        out_shape=jax.ShapeDtypeStruct((nw, pack, d, band), jnp.float32),
    )


def kernel(mg_emb, var_indices, patch_idx, W, b):
    n_var, n_nodes, C = mg_emb.shape
    B, V = var_indices.shape
    P = patch_idx.shape[-1]
    D = W.shape[-1]
    n_rows = B * V * P

    flat = jax.lax.optimization_barrier(mg_emb.reshape(-1))
    table = flat.reshape(n_var * n_nodes, C)
    idx = (
        var_indices.astype(jnp.int32)[:, :, None] * n_nodes
        + patch_idx.astype(jnp.int32)[:, None, :]
    ).reshape(n_rows // _CHUNK, _CHUNK)

    packed = _make_gather(n_var * n_nodes, n_rows, C)(table, idx)
    nw = 32
    band = n_rows // nw // (128 // C)
    ot = _make_proj(nw, band, C, D)(packed, W.T, b.reshape(D, 1))
    out = ot.reshape(B, V, 1, D, P)
    return jnp.swapaxes(out, -1, -2)


# speedup vs baseline: 5.3477x; 1.3669x over previous
import functools

import jax
import jax.numpy as jnp
from jax import lax
from jax.experimental import pallas as pl
from jax.experimental.pallas import tpu as pltpu
from jax.experimental.pallas import tpu_sc as plsc

_CHUNK = 128
_NBUF = 8


@functools.lru_cache(maxsize=None)
def _make_gather(n_table_rows: int, n_rows: int, row_width: int):
    info = plsc.get_sparse_core_info()
    nc, ns = info.num_cores, info.num_subcores
    nw = nc * ns
    assert n_rows % (nw * _CHUNK) == 0
    rows_per_w = n_rows // nw
    nch = rows_per_w // _CHUNK
    ngroups = nch // _NBUF
    assert ngroups * _NBUF == nch
    pack = 128 // row_width
    band = rows_per_w // pack
    chunks_per_u = band // _CHUNK
    mesh = plsc.VectorSubcoreMesh(core_axis_name="c", subcore_axis_name="s")

    @functools.partial(
        pl.kernel,
        mesh=mesh,
        out_type=jax.ShapeDtypeStruct((n_rows // pack, 128), jnp.float32),
        scratch_types=[
            pltpu.VMEM((nch, _CHUNK), jnp.int32),
            pltpu.VMEM((_NBUF, _CHUNK, row_width), jnp.float32),
            pltpu.SemaphoreType.DMA,
            pltpu.SemaphoreType.DMA,
        ],
        compiler_params=pltpu.CompilerParams(use_tc_tiling_on_sc=False),
    )
    def gather(table_hbm, idx_hbm, out_hbm, idx_v, rows_v, gsem, wsem):
        wid = lax.axis_index("s") * nc + lax.axis_index("c")
        pltpu.sync_copy(idx_hbm.at[pl.ds(wid * nch, nch)], idx_v)
        base = wid * band

        def group(g, _):
            j0 = g * _NBUF
            for bb in range(_NBUF):
                pltpu.async_copy(table_hbm.at[idx_v.at[j0 + bb]], rows_v.at[bb], gsem)
            for bb in range(_NBUF):
                pltpu.make_async_copy(table_hbm.at[idx_v.at[j0 + bb]], rows_v.at[bb], gsem).wait()
            for bb in range(_NBUF):
                c = j0 + bb
                dst = out_hbm.at[
                    pl.ds(base + (c % chunks_per_u) * _CHUNK, _CHUNK),
                    pl.ds((c // chunks_per_u) * row_width, row_width),
                ]
                pltpu.async_copy(rows_v.at[bb], dst, wsem)
            for bb in range(_NBUF):
                c = j0 + bb
                dst = out_hbm.at[
                    pl.ds(base + (c % chunks_per_u) * _CHUNK, _CHUNK),
                    pl.ds((c // chunks_per_u) * row_width, row_width),
                ]
                pltpu.make_async_copy(rows_v.at[bb], dst, wsem).wait()
            return 0

        lax.fori_loop(0, ngroups, group, 0)

    return gather


_TCH = 5120
_TK = 5


def _t_body(src_hbm, tail_ref, o_ref, xbuf, sem, *, n_var, nodes, ch, K):
    CH = _TCH
    Q = K * CH
    v = pl.program_id(0)
    k = pl.program_id(1)
    max_off = ((nodes - CH) // 128) * 128
    k_tail = (nodes - 3 * Q) // CH
    tail_base = 3 * Q + k_tail * CH

    def dma(vv, par):
        return pltpu.make_async_copy(src_hbm.at[vv], xbuf.at[par], sem.at[par])

    @pl.when((v == 0) & (k == 0))
    def _prologue():
        dma(0, 0).start()

    @pl.when(k == 0)
    def _head():
        dma(v, v % 2).wait()

        @pl.when(v + 1 < n_var)
        def _prefetch():
            dma(v + 1, (v + 1) % 2).start()

    par = v % 2

    def tr(off):
        return jnp.swapaxes(xbuf[par, :, pl.ds(off, CH)], 0, 1)

    parts = [tr(pl.multiple_of(q * Q + k * CH, 128)) for q in range(3)]
    off3 = pl.multiple_of(
        jnp.minimum(3 * Q + k * CH, max_off).astype(jnp.int32), 128
    )

    @pl.when(k < k_tail)
    def _main_quarter3():
        o_ref[...] = jnp.concatenate(parts + [tr(off3)], axis=1)

    @pl.when(k >= k_tail)
    def _tail_quarter3():
        toff = pl.multiple_of(
            jnp.maximum(k - k_tail, 0).astype(jnp.int32) * CH, 128
        )
        o_ref[...] = jnp.concatenate(
            parts + [tail_ref[0, pl.ds(toff, CH), :]], axis=1
        )


@functools.lru_cache(maxsize=None)
def _make_transpose(n_var: int, nodes: int, ch: int):
    K = _TK
    CH = _TCH
    NP = 4 * K * CH
    assert nodes <= NP and ch * 4 == 128
    ntail_rows = _tail_rows(nodes)
    body = functools.partial(_t_body, n_var=n_var, nodes=nodes, ch=ch, K=K)
    return pl.pallas_call(
        body,
        grid=(n_var, K),
        in_specs=[
            pl.BlockSpec(memory_space=pl.ANY),
            pl.BlockSpec((1, ntail_rows, ch), lambda v, k: (v, 0, 0)),
        ],
        out_specs=pl.BlockSpec((CH, 4 * ch), lambda v, k: (v * K + k, 0)),
        out_shape=jax.ShapeDtypeStruct((n_var * NP // 4, 4 * ch), jnp.float32),
        scratch_shapes=[
            pltpu.VMEM((2, ch, nodes), jnp.float32),
            pltpu.SemaphoreType.DMA((2,)),
        ],
    )


def _tail_rows(nodes: int) -> int:
    K, CH = _TK, _TCH
    Q = K * CH
    k_tail = (nodes - 3 * Q) // CH
    return (K - k_tail) * CH


def _proj_body(x_ref, wt_ref, b_ref, o_ref):
    pack = x_ref.shape[1] // wt_ref.shape[1]
    c = wt_ref.shape[1]
    for u in range(pack):
        xu = x_ref[:, u * c:(u + 1) * c]
        ot = jax.lax.dot_general(
            wt_ref[...], xu,
            dimension_numbers=(((1,), (1,)), ((), ())),
            preferred_element_type=jnp.float32,
        )
        o_ref[0, u] = ot + b_ref[...]


@functools.lru_cache(maxsize=None)
def _make_proj(nw: int, band: int, c: int, d: int):
    pack = 128 // c
    return pl.pallas_call(
        _proj_body,
        grid=(nw,),
        in_specs=[
            pl.BlockSpec((band, 128), lambda i: (i, 0)),
            pl.BlockSpec((d, c), lambda i: (0, 0)),
            pl.BlockSpec((d, 1), lambda i: (0, 0)),
        ],
        out_specs=pl.BlockSpec((1, pack, d, band), lambda i: (i, 0, 0, 0)),
        out_shape=jax.ShapeDtypeStruct((nw, pack, d, band), jnp.float32),
    )


def kernel(mg_emb, var_indices, patch_idx, W, b):
    n_var, n_nodes, C = mg_emb.shape
    B, V = var_indices.shape
    P = patch_idx.shape[-1]
    D = W.shape[-1]
    n_rows = B * V * P

    NP = 4 * _TK * _TCH
    Qp = NP // 4
    mgT = jnp.swapaxes(mg_emb, 1, 2)
    k_tail = (n_nodes - 3 * Qp) // _TCH
    tail_base = 3 * Qp + k_tail * _TCH
    ntail = _tail_rows(n_nodes)
    tailbuf = jnp.pad(
        mg_emb[:, tail_base:, :], ((0, 0), (0, ntail - (n_nodes - tail_base)), (0, 0))
    )
    t128 = _make_transpose(n_var, n_nodes, C)(mgT, tailbuf)
    table = t128.reshape(n_var * NP, C)

    pq = patch_idx.astype(jnp.int32)
    perm = 4 * (pq % Qp) + pq // Qp
    idx = (
        var_indices.astype(jnp.int32)[:, :, None] * NP + perm[:, None, :]
    ).reshape(n_rows // _CHUNK, _CHUNK)

    packed = _make_gather(n_var * NP, n_rows, C)(table, idx)
    nw = 32
    band = n_rows // nw // (128 // C)
    ot = _make_proj(nw, band, C, D)(packed, W.T, b.reshape(D, 1))
    out = ot.reshape(B, V, 1, D, P)
    return jnp.swapaxes(out, -1, -2)
